# baseline (speedup 1.0000x reference)
import functools

import jax
import jax.numpy as jnp
from jax import lax
from jax.experimental import pallas as pl
from jax.experimental.pallas import tpu as pltpu
from jax.experimental.pallas import tpu_sc as plsc

N = 10000
D = 128
KN = 16
NP = 10240
RB = 256
CB = 2048
NCHUNK = NP // CB
BN = 512
NEG = -3.0e38

NW = 32
GCHUNK = 128
NG = 4
GR = NP // NG
PER_W = KN * GR // NW


def _normalize_body(f_ref, x_ref):
    f = f_ref[...]
    n = jnp.sqrt(jnp.sum(f * f, axis=1, keepdims=True))
    x_ref[...] = f / jnp.maximum(n, 1e-12)


def _topk_body(xr_ref, x_ref, idx_ref):
    xr = xr_ref[...]
    sims = lax.dot_general(xr, x_ref[...], (((1,), (1,)), ((), ())),
                           preferred_element_type=jnp.float32)
    iota = lax.broadcasted_iota(jnp.int32, (RB, NP), 1)
    sims = jnp.where(iota < N, sims, NEG)
    for t in range(KN):
        am = jnp.argmax(sims, axis=1)[:, None]
        idx_ref[:, t:t + 1] = am.astype(jnp.int32)
        if t + 1 < KN:
            sims = jnp.where(iota == am, NEG, sims)


def _vconv_body(near_ref, wkk_ref, bkk_ref, wk1_ref, bk1_ref, wfc_ref,
                bfc_ref, out_ref):
    w = jnp.zeros((BN, KN), jnp.float32)
    for a in range(KN):
        na = near_ref[a]
        mult = lax.dot_general(na, wkk_ref[a], (((1,), (1,)), ((), ())),
                               preferred_element_type=jnp.float32,
                               precision=lax.Precision.HIGHEST)
        mult = mult + bkk_ref[a]
        m = jnp.max(mult, axis=1, keepdims=True)
        e = jnp.exp(mult - m)
        soft = e / jnp.sum(e, axis=1, keepdims=True)
        w = w + wk1_ref[:, a:a + 1] * soft
    pooled = w[:, 0:1] * near_ref[0]
    for b in range(1, KN):
        pooled = pooled + w[:, b:b + 1] * near_ref[b]
    pooled = pooled + bk1_ref[:, 0:1]
    out = lax.dot_general(pooled, wfc_ref[...], (((1,), (1,)), ((), ())),
                          preferred_element_type=jnp.float32,
                          precision=lax.Precision.HIGHEST)
    out_ref[...] = out + bfc_ref[...]


def _make_gather():
    mesh = plsc.VectorSubcoreMesh(core_axis_name="c", subcore_axis_name="s")
    nch = PER_W // GCHUNK

    @functools.partial(
        pl.kernel, mesh=mesh,
        out_type=jax.ShapeDtypeStruct((KN * GR, D), jnp.float32),
        scratch_types=[
            pltpu.VMEM((nch, GCHUNK), jnp.int32),
            pltpu.VMEM((2, GCHUNK, D), jnp.float32),
            pltpu.SemaphoreType.DMA,
            pltpu.SemaphoreType.DMA,
        ],
    )
    def gk(idx_hbm, table_hbm, out_hbm, idx_v, rows_v, gsem, osem):
        wid = lax.axis_index("s") * 2 + lax.axis_index("c")
        base = wid * PER_W
        pltpu.sync_copy(idx_hbm.at[wid], idx_v)

        def gstart(c, b):
            pltpu.async_copy(table_hbm.at[idx_v.at[c]], rows_v.at[b], gsem)

        def ostart(c, b):
            pltpu.async_copy(rows_v.at[b],
                             out_hbm.at[pl.ds(base + c * GCHUNK, GCHUNK)],
                             osem)

        gstart(0, 0)
        gstart(1, 1)

        def body(i, carry):
            c = 2 * i
            for b in range(2):
                pltpu.make_async_copy(table_hbm.at[idx_v.at[c + b]],
                                      rows_v.at[b], gsem).wait()
                ostart(c + b, b)
                pltpu.make_async_copy(rows_v.at[b], out_hbm.at[
                    pl.ds(base + (c + b) * GCHUNK, GCHUNK)], osem).wait()
                nxt = c + b + 2

                @pl.when(nxt < nch)
                def _():
                    gstart(nxt, b)
            return carry

        lax.fori_loop(0, nch // 2, body, 0)

    return gk


def kernel(feats, edge_dict, ite, Wkk, bkk, Wk1, bk1, W1, b1, W2, b2, Wfc,
           bfc):
    feats_p = jnp.pad(feats, ((0, NP - N), (0, 0)))

    x = pl.pallas_call(
        _normalize_body,
        out_shape=jax.ShapeDtypeStruct((NP, D), jnp.float32),
    )(feats_p)

    gather = _make_gather()
    outs = []
    for g in range(NG):
        xg = lax.slice(x, (g * GR, 0), ((g + 1) * GR, D))
        idx_g = pl.pallas_call(
            _topk_body,
            grid=(GR // RB,),
            in_specs=[
                pl.BlockSpec((RB, D), lambda i: (i, 0)),
                pl.BlockSpec((NP, D), lambda i: (0, 0)),
            ],
            out_specs=pl.BlockSpec((RB, KN), lambda i: (i, 0)),
            out_shape=jax.ShapeDtypeStruct((GR, KN), jnp.int32),
        )(xg, x)

        idx_t = idx_g.T.reshape(NW, PER_W // GCHUNK, GCHUNK)
        near3 = gather(idx_t, feats).reshape(KN, GR, D)

        out_g = pl.pallas_call(
            _vconv_body,
            grid=(GR // BN,),
            in_specs=[
                pl.BlockSpec((KN, BN, D), lambda i: (0, i, 0)),
                pl.BlockSpec((KN, KN, D), lambda i: (0, 0, 0)),
                pl.BlockSpec((KN, 1, KN), lambda i: (0, 0, 0)),
                pl.BlockSpec((1, KN), lambda i: (0, 0)),
                pl.BlockSpec((1, 1), lambda i: (0, 0)),
                pl.BlockSpec((D, D), lambda i: (0, 0)),
                pl.BlockSpec((1, D), lambda i: (0, 0)),
            ],
            out_specs=pl.BlockSpec((BN, D), lambda i: (i, 0)),
            out_shape=jax.ShapeDtypeStruct((GR, D), jnp.float32),
        )(near3, Wkk.reshape(KN, KN, D), bkk.reshape(KN, 1, KN),
          Wk1.reshape(1, KN), bk1.reshape(1, 1), Wfc, bfc.reshape(1, D))
        outs.append(out_g)

    return jnp.concatenate(outs, axis=0)[:N]


# speedup vs baseline: 1.0749x; 1.0749x over previous
import functools

import jax
import jax.numpy as jnp
from jax import lax
from jax.experimental import pallas as pl
from jax.experimental.pallas import tpu as pltpu
from jax.experimental.pallas import tpu_sc as plsc

N = 10000
D = 128
KN = 16
NP = 10240
RB = 256
CB = 2048
NCHUNK = NP // CB
BN = 512
NEG = -3.0e38

NW = 32
GCHUNK = 128
NG = 1
GR = NP // NG
PER_W = KN * GR // NW


def _normalize_body(f_ref, x_ref):
    f = f_ref[...]
    n = jnp.sqrt(jnp.sum(f * f, axis=1, keepdims=True))
    x_ref[...] = f / jnp.maximum(n, 1e-12)


def _topk_body(xr_ref, x_ref, idx_ref):
    xr = xr_ref[...]
    sims = lax.dot_general(xr, x_ref[...], (((1,), (1,)), ((), ())),
                           preferred_element_type=jnp.float32)
    iota = lax.broadcasted_iota(jnp.int32, (RB, NP), 1)
    sims = jnp.where(iota < N, sims, NEG)
    row_ids = (pl.program_id(0) * RB
               + lax.broadcasted_iota(jnp.int32, (RB, 1), 0))
    idx_ref[:, 0:1] = row_ids
    sims = jnp.where(iota == row_ids, NEG, sims)
    for t in range(1, KN):
        am = jnp.argmax(sims, axis=1)[:, None]
        idx_ref[:, t:t + 1] = am
        if t + 1 < KN:
            sims = jnp.where(iota == am, NEG, sims)


def _vconv_body(near_ref, wkk_ref, bkk_ref, wk1_ref, bk1_ref, wfc_ref,
                bfc_ref, out_ref):
    w = jnp.zeros((BN, KN), jnp.float32)
    for a in range(KN):
        na = near_ref[a]
        mult = lax.dot_general(na, wkk_ref[a], (((1,), (1,)), ((), ())),
                               preferred_element_type=jnp.float32,
                               precision=lax.Precision.HIGHEST)
        mult = mult + bkk_ref[a]
        m = jnp.max(mult, axis=1, keepdims=True)
        e = jnp.exp(mult - m)
        soft = e / jnp.sum(e, axis=1, keepdims=True)
        w = w + wk1_ref[:, a:a + 1] * soft
    pooled = w[:, 0:1] * near_ref[0]
    for b in range(1, KN):
        pooled = pooled + w[:, b:b + 1] * near_ref[b]
    pooled = pooled + bk1_ref[:, 0:1]
    out = lax.dot_general(pooled, wfc_ref[...], (((1,), (1,)), ((), ())),
                          preferred_element_type=jnp.float32,
                          precision=lax.Precision.HIGHEST)
    out_ref[...] = out + bfc_ref[...]


def _make_gather():
    mesh = plsc.VectorSubcoreMesh(core_axis_name="c", subcore_axis_name="s")
    nch = PER_W // GCHUNK

    @functools.partial(
        pl.kernel, mesh=mesh,
        out_type=jax.ShapeDtypeStruct((KN * GR, D), jnp.float32),
        scratch_types=[
            pltpu.VMEM((nch, GCHUNK), jnp.int32),
            pltpu.VMEM((2, GCHUNK, D), jnp.float32),
            pltpu.SemaphoreType.DMA,
            pltpu.SemaphoreType.DMA,
        ],
    )
    def gk(idx_hbm, table_hbm, out_hbm, idx_v, rows_v, gsem, osem):
        wid = lax.axis_index("s") * 2 + lax.axis_index("c")
        base = wid * PER_W
        pltpu.sync_copy(idx_hbm.at[wid], idx_v)

        def gstart(c, b):
            pltpu.async_copy(table_hbm.at[idx_v.at[c]], rows_v.at[b], gsem)

        def ostart(c, b):
            pltpu.async_copy(rows_v.at[b],
                             out_hbm.at[pl.ds(base + c * GCHUNK, GCHUNK)],
                             osem)

        gstart(0, 0)
        gstart(1, 1)

        def body(i, carry):
            c = 2 * i
            for b in range(2):
                pltpu.make_async_copy(table_hbm.at[idx_v.at[c + b]],
                                      rows_v.at[b], gsem).wait()
                ostart(c + b, b)
                pltpu.make_async_copy(rows_v.at[b], out_hbm.at[
                    pl.ds(base + (c + b) * GCHUNK, GCHUNK)], osem).wait()
                nxt = c + b + 2

                @pl.when(nxt < nch)
                def _():
                    gstart(nxt, b)
            return carry

        lax.fori_loop(0, nch // 2, body, 0)

    return gk


def kernel(feats, edge_dict, ite, Wkk, bkk, Wk1, bk1, W1, b1, W2, b2, Wfc,
           bfc):
    feats_p = jnp.pad(feats, ((0, NP - N), (0, 0)))

    x = pl.pallas_call(
        _normalize_body,
        out_shape=jax.ShapeDtypeStruct((NP, D), jnp.float32),
    )(feats_p)

    gather = _make_gather()
    outs = []
    for g in range(NG):
        xg = lax.slice(x, (g * GR, 0), ((g + 1) * GR, D))
        idx_g = pl.pallas_call(
            _topk_body,
            grid=(GR // RB,),
            in_specs=[
                pl.BlockSpec((RB, D), lambda i: (i, 0)),
                pl.BlockSpec((NP, D), lambda i: (0, 0)),
            ],
            out_specs=pl.BlockSpec((RB, KN), lambda i: (i, 0)),
            out_shape=jax.ShapeDtypeStruct((GR, KN), jnp.int32),
        )(xg, x)

        idx_t = idx_g.T.reshape(NW, PER_W // GCHUNK, GCHUNK)
        near3 = gather(idx_t, feats).reshape(KN, GR, D)

        out_g = pl.pallas_call(
            _vconv_body,
            grid=(GR // BN,),
            in_specs=[
                pl.BlockSpec((KN, BN, D), lambda i: (0, i, 0)),
                pl.BlockSpec((KN, KN, D), lambda i: (0, 0, 0)),
                pl.BlockSpec((KN, 1, KN), lambda i: (0, 0, 0)),
                pl.BlockSpec((1, KN), lambda i: (0, 0)),
                pl.BlockSpec((1, 1), lambda i: (0, 0)),
                pl.BlockSpec((D, D), lambda i: (0, 0)),
                pl.BlockSpec((1, D), lambda i: (0, 0)),
            ],
            out_specs=pl.BlockSpec((BN, D), lambda i: (i, 0)),
            out_shape=jax.ShapeDtypeStruct((GR, D), jnp.float32),
        )(near3, Wkk.reshape(KN, KN, D), bkk.reshape(KN, 1, KN),
          Wk1.reshape(1, KN), bk1.reshape(1, 1), Wfc, bfc.reshape(1, D))
        outs.append(out_g)

    return jnp.concatenate(outs, axis=0)[:N]


# speedup vs baseline: 1.1635x; 1.0824x over previous
import functools

import jax
import jax.numpy as jnp
from jax import lax
from jax.experimental import pallas as pl
from jax.experimental.pallas import tpu as pltpu
from jax.experimental.pallas import tpu_sc as plsc

N = 10000
D = 128
KN = 16
NP = 10240
RB = 512
CB = 2048
NCHUNK = NP // CB
BN = 512
NEG = -3.0e38

NW = 32
GCHUNK = 128
NG = 1
GR = NP // NG
PER_W = KN * GR // NW


def _normalize_body(f_ref, x_ref):
    f = f_ref[...]
    n = jnp.sqrt(jnp.sum(f * f, axis=1, keepdims=True))
    x_ref[...] = f / jnp.maximum(n, 1e-12)


def _topk_body(xr_ref, x_ref, idx_ref):
    xr = xr_ref[...]
    sims = lax.dot_general(xr, x_ref[...], (((1,), (1,)), ((), ())),
                           preferred_element_type=jnp.float32)
    iota = lax.broadcasted_iota(jnp.int32, (RB, NP), 1)
    sims = jnp.where(iota < N, sims, NEG)
    row_ids = (pl.program_id(0) * RB
               + lax.broadcasted_iota(jnp.int32, (RB, 1), 0))
    idx_ref[:, 0:1] = row_ids
    sims = jnp.where(iota == row_ids, NEG, sims)
    for t in range(1, KN):
        am = jnp.argmax(sims, axis=1)[:, None]
        idx_ref[:, t:t + 1] = am
        if t + 1 < KN:
            sims = jnp.where(iota == am, NEG, sims)


def _vconv_body(near_ref, wkk_ref, bkk_ref, wk1_ref, bk1_ref, wfc_ref,
                bfc_ref, out_ref):
    w = jnp.zeros((BN, KN), jnp.float32)
    for a in range(KN):
        na = near_ref[a]
        mult = lax.dot_general(na, wkk_ref[a], (((1,), (1,)), ((), ())),
                               preferred_element_type=jnp.float32,
                               precision=lax.Precision.HIGHEST)
        mult = mult + bkk_ref[a]
        m = jnp.max(mult, axis=1, keepdims=True)
        e = jnp.exp(mult - m)
        soft = e / jnp.sum(e, axis=1, keepdims=True)
        w = w + wk1_ref[:, a:a + 1] * soft
    pooled = w[:, 0:1] * near_ref[0]
    for b in range(1, KN):
        pooled = pooled + w[:, b:b + 1] * near_ref[b]
    pooled = pooled + bk1_ref[:, 0:1]
    out = lax.dot_general(pooled, wfc_ref[...], (((1,), (1,)), ((), ())),
                          preferred_element_type=jnp.float32,
                          precision=lax.Precision.HIGHEST)
    out_ref[...] = out + bfc_ref[...]


def _make_gather():
    mesh = plsc.VectorSubcoreMesh(core_axis_name="c", subcore_axis_name="s")
    nch = PER_W // GCHUNK

    @functools.partial(
        pl.kernel, mesh=mesh,
        out_type=jax.ShapeDtypeStruct((KN * GR, D), jnp.float32),
        scratch_types=[
            pltpu.VMEM((nch, GCHUNK), jnp.int32),
            pltpu.VMEM((2, GCHUNK, D), jnp.float32),
            pltpu.SemaphoreType.DMA,
            pltpu.SemaphoreType.DMA,
        ],
    )
    def gk(idx_hbm, table_hbm, out_hbm, idx_v, rows_v, gsem, osem):
        wid = lax.axis_index("s") * 2 + lax.axis_index("c")
        base = wid * PER_W
        pltpu.sync_copy(idx_hbm.at[wid], idx_v)

        def gstart(c, b):
            pltpu.async_copy(table_hbm.at[idx_v.at[c]], rows_v.at[b], gsem)

        def ostart(c, b):
            pltpu.async_copy(rows_v.at[b],
                             out_hbm.at[pl.ds(base + c * GCHUNK, GCHUNK)],
                             osem)

        gstart(0, 0)
        gstart(1, 1)

        def body(i, carry):
            c = 2 * i
            for b in range(2):
                pltpu.make_async_copy(table_hbm.at[idx_v.at[c + b]],
                                      rows_v.at[b], gsem).wait()
                ostart(c + b, b)
                pltpu.make_async_copy(rows_v.at[b], out_hbm.at[
                    pl.ds(base + (c + b) * GCHUNK, GCHUNK)], osem).wait()
                nxt = c + b + 2

                @pl.when(nxt < nch)
                def _():
                    gstart(nxt, b)
            return carry

        lax.fori_loop(0, nch // 2, body, 0)

    return gk


def kernel(feats, edge_dict, ite, Wkk, bkk, Wk1, bk1, W1, b1, W2, b2, Wfc,
           bfc):
    feats_p = jnp.pad(feats, ((0, NP - N), (0, 0)))

    x = pl.pallas_call(
        _normalize_body,
        out_shape=jax.ShapeDtypeStruct((NP, D), jnp.float32),
    )(feats_p)

    gather = _make_gather()
    outs = []
    for g in range(NG):
        xg = lax.slice(x, (g * GR, 0), ((g + 1) * GR, D))
        idx_g = pl.pallas_call(
            _topk_body,
            grid=(GR // RB,),
            in_specs=[
                pl.BlockSpec((RB, D), lambda i: (i, 0)),
                pl.BlockSpec((NP, D), lambda i: (0, 0)),
            ],
            out_specs=pl.BlockSpec((RB, KN), lambda i: (i, 0)),
            out_shape=jax.ShapeDtypeStruct((GR, KN), jnp.int32),
        )(xg, x)

        idx_t = idx_g.T.reshape(NW, PER_W // GCHUNK, GCHUNK)
        near3 = gather(idx_t, feats).reshape(KN, GR, D)

        out_g = pl.pallas_call(
            _vconv_body,
            grid=(GR // BN,),
            in_specs=[
                pl.BlockSpec((KN, BN, D), lambda i: (0, i, 0)),
                pl.BlockSpec((KN, KN, D), lambda i: (0, 0, 0)),
                pl.BlockSpec((KN, 1, KN), lambda i: (0, 0, 0)),
                pl.BlockSpec((1, KN), lambda i: (0, 0)),
                pl.BlockSpec((1, 1), lambda i: (0, 0)),
                pl.BlockSpec((D, D), lambda i: (0, 0)),
                pl.BlockSpec((1, D), lambda i: (0, 0)),
            ],
            out_specs=pl.BlockSpec((BN, D), lambda i: (i, 0)),
            out_shape=jax.ShapeDtypeStruct((GR, D), jnp.float32),
        )(near3, Wkk.reshape(KN, KN, D), bkk.reshape(KN, 1, KN),
          Wk1.reshape(1, KN), bk1.reshape(1, 1), Wfc, bfc.reshape(1, D))
        outs.append(out_g)

    return jnp.concatenate(outs, axis=0)[:N]


# speedup vs baseline: 1.1636x; 1.0001x over previous
import functools

import jax
import jax.numpy as jnp
from jax import lax
from jax.experimental import pallas as pl
from jax.experimental.pallas import tpu as pltpu
from jax.experimental.pallas import tpu_sc as plsc

N = 10000
D = 128
KN = 16
NP = 10240
RB = 512
BN = 512
NEG = -3.0e38

NW = 32
GCHUNK = 128
NG = 1
GR = NP // NG
PER_W = KN * GR // NW


def _normalize_body(f_ref, x_ref):
    f = f_ref[...]
    n = jnp.sqrt(jnp.sum(f * f, axis=1, keepdims=True))
    x_ref[...] = f / jnp.maximum(n, 1e-12)


def _topk_body(xr_ref, x_ref, idx_ref):
    xr = xr_ref[...]
    sims = lax.dot_general(xr, x_ref[...], (((1,), (1,)), ((), ())),
                           preferred_element_type=jnp.float32)
    iota = lax.broadcasted_iota(jnp.int32, (RB, NP), 1)
    sims = jnp.where(iota < N, sims, NEG)
    row_ids = (pl.program_id(0) * RB
               + lax.broadcasted_iota(jnp.int32, (RB, 1), 0))
    idx_ref[:, 0:1] = row_ids
    sims = jnp.where(iota == row_ids, NEG, sims)
    for t in range(1, KN):
        am = jnp.argmax(sims, axis=1)[:, None]
        idx_ref[:, t:t + 1] = am
        if t + 1 < KN:
            sims = jnp.where(iota == am, NEG, sims)


def _vconv_body(near_ref, wkk_ref, bkk_ref, wk1_ref, bk1_ref, wfc_ref,
                bfc_ref, out_ref):
    w = jnp.zeros((BN, KN), jnp.float32)
    for a in range(KN):
        na = near_ref[a]
        mult = lax.dot_general(na, wkk_ref[a], (((1,), (1,)), ((), ())),
                               preferred_element_type=jnp.float32,
                               precision=lax.Precision.HIGHEST)
        mult = mult + bkk_ref[a]
        m = jnp.max(mult, axis=1, keepdims=True)
        e = jnp.exp(mult - m)
        soft = e / jnp.sum(e, axis=1, keepdims=True)
        w = w + wk1_ref[:, a:a + 1] * soft
    pooled = w[:, 0:1] * near_ref[0]
    for b in range(1, KN):
        pooled = pooled + w[:, b:b + 1] * near_ref[b]
    pooled = pooled + bk1_ref[:, 0:1]
    out = lax.dot_general(pooled, wfc_ref[...], (((1,), (1,)), ((), ())),
                          preferred_element_type=jnp.float32,
                          precision=lax.Precision.HIGHEST)
    out_ref[...] = out + bfc_ref[...]


def _make_gather():
    mesh = plsc.VectorSubcoreMesh(core_axis_name="c", subcore_axis_name="s")
    nch = PER_W // GCHUNK

    @functools.partial(
        pl.kernel, mesh=mesh,
        out_type=jax.ShapeDtypeStruct((KN * GR, D), jnp.float32),
        scratch_types=[
            pltpu.VMEM((nch, GCHUNK), jnp.int32),
            pltpu.VMEM((2, GCHUNK, D), jnp.float32),
            pltpu.SemaphoreType.DMA,
            pltpu.SemaphoreType.DMA,
        ],
    )
    def gk(idx_hbm, table_hbm, out_hbm, idx_v, rows_v, gsem, osem):
        wid = lax.axis_index("s") * 2 + lax.axis_index("c")
        base = wid * PER_W
        pltpu.sync_copy(idx_hbm.at[wid], idx_v)

        def gstart(c, b):
            pltpu.async_copy(table_hbm.at[idx_v.at[c]], rows_v.at[b], gsem)

        def ostart(c, b):
            pltpu.async_copy(rows_v.at[b],
                             out_hbm.at[pl.ds(base + c * GCHUNK, GCHUNK)],
                             osem)

        gstart(0, 0)
        gstart(1, 1)

        def body(i, carry):
            c = 2 * i
            for b in range(2):
                pltpu.make_async_copy(table_hbm.at[idx_v.at[c + b]],
                                      rows_v.at[b], gsem).wait()
                ostart(c + b, b)
                pltpu.make_async_copy(rows_v.at[b], out_hbm.at[
                    pl.ds(base + (c + b) * GCHUNK, GCHUNK)], osem).wait()
                nxt = c + b + 2

                @pl.when(nxt < nch)
                def _():
                    gstart(nxt, b)
            return carry

        lax.fori_loop(0, nch // 2, body, 0)

    return gk


def kernel(feats, edge_dict, ite, Wkk, bkk, Wk1, bk1, W1, b1, W2, b2, Wfc,
           bfc):
    feats_p = jnp.pad(feats, ((0, NP - N), (0, 0)))

    x = pl.pallas_call(
        _normalize_body,
        out_shape=jax.ShapeDtypeStruct((NP, D), jnp.float32),
    )(feats_p)

    gather = _make_gather()
    outs = []
    for g in range(NG):
        xg = lax.slice(x, (g * GR, 0), ((g + 1) * GR, D))
        idx_g = pl.pallas_call(
            _topk_body,
            grid=(GR // RB,),
            in_specs=[
                pl.BlockSpec((RB, D), lambda i: (i, 0)),
                pl.BlockSpec((NP, D), lambda i: (0, 0)),
            ],
            out_specs=pl.BlockSpec((RB, KN), lambda i: (i, 0)),
            out_shape=jax.ShapeDtypeStruct((GR, KN), jnp.int32),
        )(xg, x)

        idx_t = idx_g.T.reshape(NW, PER_W // GCHUNK, GCHUNK)
        near3 = gather(idx_t, feats).reshape(KN, GR, D)

        out_g = pl.pallas_call(
            _vconv_body,
            grid=(GR // BN,),
            in_specs=[
                pl.BlockSpec((KN, BN, D), lambda i: (0, i, 0)),
                pl.BlockSpec((KN, KN, D), lambda i: (0, 0, 0)),
                pl.BlockSpec((KN, 1, KN), lambda i: (0, 0, 0)),
                pl.BlockSpec((1, KN), lambda i: (0, 0)),
                pl.BlockSpec((1, 1), lambda i: (0, 0)),
                pl.BlockSpec((D, D), lambda i: (0, 0)),
                pl.BlockSpec((1, D), lambda i: (0, 0)),
            ],
            out_specs=pl.BlockSpec((BN, D), lambda i: (i, 0)),
            out_shape=jax.ShapeDtypeStruct((GR, D), jnp.float32),
        )(near3, Wkk.reshape(KN, KN, D), bkk.reshape(KN, 1, KN),
          Wk1.reshape(1, KN), bk1.reshape(1, 1), Wfc, bfc.reshape(1, D))
        outs.append(out_g)

    return jnp.concatenate(outs, axis=0)[:N]
